# Initial kernel scaffold; baseline (speedup 1.0000x reference)
import jax
import jax.numpy as jnp
from jax.experimental import pallas as pl


def kernel(x, neighbors, weights):
    raise NotImplementedError("write your pallas kernel here")


# speedup vs baseline: 1.1209x; 1.1209x over previous
import functools

import jax
import jax.numpy as jnp
from jax import lax
from jax.experimental import pallas as pl
from jax.experimental.pallas import tpu as pltpu
from jax.experimental.pallas import tpu_sc as plsc

N = 10000
K = 16
D = 256
LANES = 16
NC = 2
NS = 16
NW = NC * NS
PER_W = 320
NPAD = NW * PER_W
B = 8
NBATCH = PER_W // B
DC = D // LANES


def _make_sc_call():
    mesh = plsc.VectorSubcoreMesh(core_axis_name="c", subcore_axis_name="s")

    @functools.partial(
        pl.kernel,
        mesh=mesh,
        compiler_params=pltpu.CompilerParams(needs_layout_passes=False),
        out_type=jax.ShapeDtypeStruct((NPAD, D), jnp.float32),
        scratch_types=[
            pltpu.VMEM((B * K,), jnp.int32),
            pltpu.VMEM((B * K, D), jnp.float32),
            pltpu.VMEM((B, K), jnp.float32),
            pltpu.VMEM((B, D), jnp.float32),
            pltpu.SemaphoreType.DMA,
        ],
    )
    def sc_kernel(x_hbm, nbr_hbm, w_hbm, out_hbm, idx_v, rows_v, w_v, out_v, sem):
        wid = lax.axis_index("s") * NC + lax.axis_index("c")

        def batch_body(g, _):
            node_base = wid * PER_W + g * B
            idx_base = pl.multiple_of(node_base * K, 128)
            pltpu.sync_copy(nbr_hbm.at[pl.ds(idx_base, B * K)], idx_v)
            pltpu.async_copy(x_hbm.at[idx_v], rows_v, sem).wait()
            pltpu.sync_copy(w_hbm.at[pl.ds(node_base, B)], w_v)

            def node_body(b, _):
                w_row = w_v[b, :]
                norm = plsc.cumsum(w_row)[K - 1]
                is0 = norm == 0.0
                safe = jnp.where(is0, jnp.float32(1.0), norm)
                wn = jnp.where(is0, jnp.full((K,), 1.0 / K, jnp.float32),
                               w_row / safe)
                row0 = b * K
                accs = [jnp.zeros((LANES,), jnp.float32) for _ in range(DC)]
                for k in range(K):
                    wk = wn[k]
                    r = row0 + k
                    for c in range(DC):
                        accs[c] = accs[c] + wk * rows_v[r, pl.ds(c * LANES, LANES)]
                for c in range(DC):
                    out_v[b, pl.ds(c * LANES, LANES)] = accs[c]
                return 0

            lax.fori_loop(0, B, node_body, 0)
            pltpu.sync_copy(out_v, out_hbm.at[pl.ds(node_base, B)])
            return 0

        lax.fori_loop(0, NBATCH, batch_body, 0)

    return sc_kernel


_sc_call = _make_sc_call()


@jax.jit
def kernel(x, neighbors, weights):
    nbr = neighbors.astype(jnp.int32)
    pad = NPAD - N
    nbr_p = jnp.pad(nbr, ((0, pad), (0, 0))).reshape(-1)
    w_p = jnp.pad(weights, ((0, pad), (0, 0)))
    out = _sc_call(x, nbr_p, w_p)
    return out[:N]


# speedup vs baseline: 1.4520x; 1.2954x over previous
import functools

import jax
import jax.numpy as jnp
from jax import lax
from jax.experimental import pallas as pl
from jax.experimental.pallas import tpu as pltpu
from jax.experimental.pallas import tpu_sc as plsc

N = 10000
K = 16
D = 256
LANES = 16
NC = 2
NS = 16
NW = NC * NS
PER_W = 320
NPAD = NW * PER_W
B = 8
NBATCH = PER_W // B
NBUF = 2
DC = D // LANES


def _make_sc_call():
    mesh = plsc.VectorSubcoreMesh(core_axis_name="c", subcore_axis_name="s")

    @functools.partial(
        pl.kernel,
        mesh=mesh,
        compiler_params=pltpu.CompilerParams(needs_layout_passes=False),
        out_type=jax.ShapeDtypeStruct((NPAD, D), jnp.float32),
        scratch_types=[
            pltpu.VMEM((PER_W * K,), jnp.int32),
            pltpu.VMEM((PER_W, K), jnp.float32),
            pltpu.VMEM((NBUF, B * K, D), jnp.float32),
            pltpu.VMEM((NBUF, B, D), jnp.float32),
            pltpu.SemaphoreType.DMA,
            pltpu.SemaphoreType.DMA,
            pltpu.SemaphoreType.DMA,
            pltpu.SemaphoreType.DMA,
        ],
    )
    def sc_kernel(x_hbm, nbr_hbm, w_hbm, out_hbm, idx_v, w_v, rows_v, out_v,
                  gsem0, gsem1, osem0, osem1):
        wid = lax.axis_index("s") * NC + lax.axis_index("c")
        node0 = wid * PER_W
        gsems = [gsem0, gsem1]
        osems = [osem0, osem1]

        pltpu.sync_copy(nbr_hbm.at[pl.ds(node0 * K, PER_W * K)], idx_v)
        pltpu.sync_copy(w_hbm.at[pl.ds(node0, PER_W)], w_v)

        def gather(t, g):
            pltpu.async_copy(
                x_hbm.at[idx_v.at[pl.ds(g * (B * K), B * K)]],
                rows_v.at[t], gsems[t])

        def gather_wait(t, g):
            pltpu.make_async_copy(
                x_hbm.at[idx_v.at[pl.ds(g * (B * K), B * K)]],
                rows_v.at[t], gsems[t]).wait()

        def out_issue(t, g):
            pltpu.async_copy(out_v.at[t],
                             out_hbm.at[pl.ds(node0 + g * B, B)], osems[t])

        def out_wait(t, g):
            pltpu.make_async_copy(out_v.at[t],
                                  out_hbm.at[pl.ds(node0 + g * B, B)],
                                  osems[t]).wait()

        for t in range(NBUF):
            gather(t, t)

        def outer(j, _):
            for t in range(NBUF):
                g = j * NBUF + t
                gather_wait(t, g)

                @pl.when(j > 0)
                def _():
                    out_wait(t, g - NBUF)

                def node_body(b, _):
                    w_row = w_v[g * B + b, :]
                    norm = plsc.cumsum(w_row)[K - 1]
                    is0 = norm == 0.0
                    safe = jnp.where(is0, jnp.float32(1.0), norm)
                    wn = jnp.where(is0, jnp.full((K,), 1.0 / K, jnp.float32),
                                   w_row / safe)
                    row0 = b * K
                    accs = [jnp.zeros((LANES,), jnp.float32)
                            for _ in range(DC)]
                    for k in range(K):
                        wk = wn[k]
                        r = row0 + k
                        for c in range(DC):
                            accs[c] = accs[c] + wk * rows_v[
                                t, r, pl.ds(c * LANES, LANES)]
                    for c in range(DC):
                        out_v[t, b, pl.ds(c * LANES, LANES)] = accs[c]
                    return 0

                lax.fori_loop(0, B, node_body, 0)
                out_issue(t, g)

                nxt = g + NBUF
                @pl.when(nxt < NBATCH)
                def _():
                    gather(t, nxt)
            return 0

        lax.fori_loop(0, NBATCH // NBUF, outer, 0)
        for t in range(NBUF):
            out_wait(t, NBATCH - NBUF + t)

    return sc_kernel


_sc_call = _make_sc_call()


@jax.jit
def kernel(x, neighbors, weights):
    nbr = neighbors.astype(jnp.int32)
    pad = NPAD - N
    nbr_p = jnp.pad(nbr, ((0, pad), (0, 0))).reshape(-1)
    w_p = jnp.pad(weights, ((0, pad), (0, 0)))
    out = _sc_call(x, nbr_p, w_p)
    return out[:N]


# speedup vs baseline: 1.7635x; 1.2145x over previous
import functools

import jax
import jax.numpy as jnp
from jax import lax
from jax.experimental import pallas as pl
from jax.experimental.pallas import tpu as pltpu
from jax.experimental.pallas import tpu_sc as plsc

N = 10000
K = 16
D = 256
LANES = 16
NC = 2
NS = 16
PAIR_W = 640
NPAD = NS * PAIR_W
W0 = 480
W1 = PAIR_W - W0
FAST_CORE = 0
B = 8
NBUF = 2
IBUF = 4
DC = D // LANES


def _make_sc_call():
    mesh = plsc.VectorSubcoreMesh(core_axis_name="c", subcore_axis_name="s")

    @functools.partial(
        pl.kernel,
        mesh=mesh,
        compiler_params=pltpu.CompilerParams(needs_layout_passes=False),
        out_type=jax.ShapeDtypeStruct((NPAD, D), jnp.float32),
        scratch_types=[
            pltpu.VMEM((IBUF, B * K), jnp.int32),
            pltpu.VMEM((IBUF, B, K), jnp.float32),
            pltpu.VMEM((NBUF, B * K, D), jnp.float32),
            pltpu.VMEM((NBUF, B, D), jnp.float32),
            pltpu.SemaphoreType.DMA,
            pltpu.SemaphoreType.DMA,
            pltpu.SemaphoreType.DMA,
            pltpu.SemaphoreType.DMA,
            pltpu.SemaphoreType.DMA,
            pltpu.SemaphoreType.DMA,
            pltpu.SemaphoreType.DMA,
            pltpu.SemaphoreType.DMA,
        ],
    )
    def sc_kernel(x_hbm, nbr_hbm, w_hbm, out_hbm, idx_v, w_v, rows_v, out_v,
                  gsem0, gsem1, osem0, osem1, isem0, isem1, isem2, isem3):
        c = lax.axis_index("c")
        s = lax.axis_index("s")
        fast = c == FAST_CORE
        node0 = s * PAIR_W + jnp.where(fast, 0, W0)
        nbatch = jnp.where(fast, W0 // B, W1 // B)
        gsems = [gsem0, gsem1]
        osems = [osem0, osem1]
        isems = [isem0, isem1, isem2, isem3]

        def stage_sync(q, g):
            pltpu.sync_copy(nbr_hbm.at[pl.ds((node0 + g * B) * K, B * K)],
                            idx_v.at[q])
            pltpu.sync_copy(w_hbm.at[pl.ds(node0 + g * B, B)], w_v.at[q])

        def stage_async(q, g):
            pltpu.async_copy(nbr_hbm.at[pl.ds((node0 + g * B) * K, B * K)],
                             idx_v.at[q], isems[q])
            pltpu.async_copy(w_hbm.at[pl.ds(node0 + g * B, B)], w_v.at[q],
                             isems[q])

        def stage_wait(q, g):
            pltpu.make_async_copy(
                nbr_hbm.at[pl.ds((node0 + g * B) * K, B * K)],
                idx_v.at[q], isems[q]).wait()
            pltpu.make_async_copy(
                w_hbm.at[pl.ds(node0 + g * B, B)], w_v.at[q],
                isems[q]).wait()

        def gather(t, q):
            pltpu.async_copy(x_hbm.at[idx_v.at[q]], rows_v.at[t], gsems[t])

        def gather_wait(t, q):
            pltpu.make_async_copy(x_hbm.at[idx_v.at[q]], rows_v.at[t],
                                  gsems[t]).wait()

        def out_issue(t, g):
            pltpu.async_copy(out_v.at[t],
                             out_hbm.at[pl.ds(node0 + g * B, B)], osems[t])

        def out_wait(t, g):
            pltpu.make_async_copy(out_v.at[t],
                                  out_hbm.at[pl.ds(node0 + g * B, B)],
                                  osems[t]).wait()

        for q in range(IBUF):
            stage_sync(q, q)
        for t in range(NBUF):
            gather(t, t)

        def outer(j, _):
            for tq in range(IBUF):
                g = j * IBUF + tq
                t = tq % NBUF
                q = tq

                gather_wait(t, q)

                @pl.when(g >= NBUF)
                def _():
                    out_wait(t, g - NBUF)

                def node_body(b, _):
                    w_row = w_v[q, b, :]
                    norm = plsc.cumsum(w_row)[K - 1]
                    is0 = norm == 0.0
                    safe = jnp.where(is0, jnp.float32(1.0), norm)
                    wn = jnp.where(is0, jnp.full((K,), 1.0 / K, jnp.float32),
                                   w_row / safe)
                    row0 = b * K
                    accs = [jnp.zeros((LANES,), jnp.float32)
                            for _ in range(DC)]
                    for k in range(K):
                        wk = wn[k]
                        r = row0 + k
                        for c_ in range(DC):
                            accs[c_] = accs[c_] + wk * rows_v[
                                t, r, pl.ds(c_ * LANES, LANES)]
                    for c_ in range(DC):
                        out_v[t, b, pl.ds(c_ * LANES, LANES)] = accs[c_]
                    return 0

                lax.fori_loop(0, B, node_body, 0)
                out_issue(t, g)

                nxt = g + NBUF
                qn = (q + NBUF) % IBUF

                @pl.when(jnp.logical_and(nxt >= IBUF, nxt < nbatch))
                def _():
                    stage_wait(qn, nxt)

                @pl.when(nxt < nbatch)
                def _():
                    gather(t, qn)

                nstage = g + IBUF

                @pl.when(nstage < nbatch)
                def _():
                    stage_async(q, nstage)
            return 0

        lax.fori_loop(0, nbatch // IBUF, outer, 0)
        for t in range(NBUF):
            out_wait(t, nbatch - NBUF + t)

    return sc_kernel


_sc_call = _make_sc_call()


@jax.jit
def kernel(x, neighbors, weights):
    nbr = neighbors.astype(jnp.int32)
    pad = NPAD - N
    nbr_p = jnp.pad(nbr, ((0, pad), (0, 0))).reshape(-1)
    w_p = jnp.pad(weights, ((0, pad), (0, 0)))
    out = _sc_call(x, nbr_p, w_p)
    return out[:N]
